# baseline (speedup 1.0000x reference)
import functools

import jax
import jax.numpy as jnp
from jax import lax
from jax.experimental import pallas as pl
from jax.experimental.pallas import tpu as pltpu
from jax.experimental.pallas import tpu_sc as plsc

B = 128
N = 32768
K = 256

NC = 2
NS = 16
L = 16
NW = NC * NS
ROWS_PER_W = B // NW
CHUNKS = N // L
NB = 256
R = 8
UN = 8
BSTR = L + 1
HIST_WORDS = NB * BSTR
HIST_TOTAL = R * HIST_WORDS
CAP = 1024
CSTR = CAP + 1


def _topbit():
    return jnp.int32(-2**31)


def _monotone(bits):
    m = lax.shift_right_arithmetic(bits, 31)
    flip = lax.bitwise_or(_topbit(), lax.bitwise_and(m, jnp.int32(0x7FFFFFFF)))
    return lax.bitwise_xor(bits, flip)


def _inverse(u):
    m = lax.shift_right_arithmetic(u, 31)
    flip = lax.bitwise_or(
        _topbit(), lax.bitwise_and(lax.bitwise_not(m), jnp.int32(0x7FFFFFFF)))
    return lax.bitwise_xor(u, flip)


def _srl(v, k):
    return lax.shift_right_logical(v, k)


def _scan_and_clear(hist_ref, tot_ref, lane):
    zero = jnp.zeros((L,), jnp.int32)
    @plsc.parallel_loop(0, NB // L, step=1, unroll=2)
    def _(c):
        base = c * L
        acc = jnp.zeros((L,), jnp.int32)
        for rep in range(R):
            for l in range(L):
                idx = rep * HIST_WORDS + (base + lane) * BSTR + l
                acc = acc + plsc.load_gather(hist_ref, [idx])
                plsc.store_scatter(hist_ref, [idx], zero)
        tot_ref[pl.ds(base, L)] = acc


def _scan_and_clear_r1(hist_ref, tot_ref, lane):
    zero = jnp.zeros((L,), jnp.int32)
    @plsc.parallel_loop(0, NB // L, step=1, unroll=2)
    def _(c):
        base = c * L
        acc = jnp.zeros((L,), jnp.int32)
        for l in range(L):
            idx = (base + lane) * BSTR + l
            acc = acc + plsc.load_gather(hist_ref, [idx])
            plsc.store_scatter(hist_ref, [idx], zero)
        tot_ref[pl.ds(base, L)] = acc


def _find2(tot_ref, kr_h, kr_l, m_h, lane):
    zv = jnp.zeros((L,), jnp.int32)
    zero = jnp.int32(0)

    def body(c, carry):
        cum_h, cum_l, bv_h, rv_h, tv_h, bv_l, rv_l, tv_l = carry
        tword = tot_ref[pl.ds(c * L, L)]
        t_h = lax.bitwise_and(tword, jnp.int32(0xFFFF))
        t_l = _srl(tword, 16)
        cs_h = cum_h + jnp.cumsum(t_h)
        cs_l = cum_l + jnp.cumsum(t_l)
        a = m_h - cs_h
        hit_h = jnp.logical_and(a < kr_h, a + t_h >= kr_h)
        cv = cs_l - t_l
        hit_l = jnp.logical_and(cv < kr_l, cv + t_l >= kr_l)
        ids = c * L + lane
        bv_h = bv_h + jnp.where(hit_h, ids + 1, zv)
        rv_h = rv_h + jnp.where(hit_h, kr_h - a, zv)
        tv_h = tv_h + jnp.where(hit_h, t_h, zv)
        bv_l = bv_l + jnp.where(hit_l, ids + 1, zv)
        rv_l = rv_l + jnp.where(hit_l, kr_l - cv, zv)
        tv_l = tv_l + jnp.where(hit_l, t_l, zv)
        nc_h = lax.squeeze(lax.slice(cs_h, (L - 1,), (L,)), (0,))
        nc_l = lax.squeeze(lax.slice(cs_l, (L - 1,), (L,)), (0,))
        return (nc_h, nc_l, bv_h, rv_h, tv_h, bv_l, rv_l, tv_l)

    out = lax.fori_loop(0, NB // L, body,
                        (zero, zero, zv, zv, zv, zv, zv, zv))
    (_, _, bv_h, rv_h, tv_h, bv_l, rv_l, tv_l) = out
    return ((jnp.sum(bv_h) - 1, jnp.sum(rv_h), jnp.sum(tv_h)),
            (jnp.sum(bv_l) - 1, jnp.sum(rv_l), jnp.sum(tv_l)))


_mesh = plsc.VectorSubcoreMesh(
    core_axis_name="c", subcore_axis_name="s", num_cores=NC, num_subcores=NS)


@functools.partial(
    pl.kernel,
    out_type=jax.ShapeDtypeStruct((B, N), jnp.float32),
    mesh=_mesh,
    compiler_params=pltpu.CompilerParams(needs_layout_passes=False),
    scratch_types=[
        pltpu.VMEM((N,), jnp.float32),
        pltpu.VMEM((N,), jnp.float32),
        pltpu.VMEM((HIST_TOTAL,), jnp.int32),
        pltpu.VMEM((NB,), jnp.int32),
        pltpu.VMEM((L * CSTR,), jnp.int32),
    ],
)
def _trunc_kernel(x_hbm, w_hbm, out_hbm, buf, w_ref, hist, tot, cand):
    wid = lax.axis_index("s") * NC + lax.axis_index("c")
    lane = lax.iota(jnp.int32, L)
    kk = jnp.int32(K)
    p1val = jnp.full((L,), 65537, jnp.int32)

    pltpu.sync_copy(w_hbm, w_ref)

    zero16 = jnp.zeros((L,), jnp.int32)
    @plsc.parallel_loop(0, HIST_TOTAL // L, step=1, unroll=8)
    def _(c):
        hist[pl.ds(c * L, L)] = zero16

    def row_body(i, _):
        r = wid * ROWS_PER_W + i
        pltpu.sync_copy(x_hbm.at[r], buf)

        @plsc.parallel_loop(0, CHUNKS, step=1, unroll=UN)
        def _(c):
            sl = pl.ds(c * L, L)
            xv = buf[sl] * w_ref[sl]
            u = _monotone(lax.bitcast_convert_type(xv, jnp.int32))
            buf[sl] = lax.bitcast_convert_type(u, jnp.float32)
            d = _srl(u, 24)
            rep = lax.bitwise_and(c + _srl(c, 3), R - 1) * HIST_WORDS
            plsc.addupdate_scatter(hist, [rep + d * BSTR + lane], p1val)

        _scan_and_clear(hist, tot, lane)
        (ph, rh, th), (plo, rl, tl) = _find2(tot, kk, kk, jnp.int32(N), lane)

        @plsc.parallel_loop(0, CHUNKS, step=1, unroll=UN,
                            carry=jnp.zeros((L,), jnp.int32))
        def off_v(c, off, ph=ph, plo=plo):
            sl = pl.ds(c * L, L)
            u = lax.bitcast_convert_type(buf[sl], jnp.int32)
            pref = _srl(u, 24)
            mh = pref == ph
            ml = pref == plo
            m = jnp.logical_or(mh, ml)
            d = lax.bitwise_and(_srl(u, 16), jnp.int32(0xFF))
            rep = lax.bitwise_and(c + _srl(c, 3), R - 1) * HIST_WORDS
            val = (jnp.where(mh, jnp.int32(1), jnp.int32(0))
                   + jnp.where(ml, jnp.int32(65536), jnp.int32(0)))
            plsc.addupdate_scatter(
                hist, [rep + d * BSTR + lane], val, mask=m)
            plsc.store_scatter(
                cand, [lane * CSTR + off], u,
                mask=jnp.logical_and(m, off < jnp.int32(CAP)))
            return off + jnp.where(m, jnp.int32(1), jnp.int32(0))

        _scan_and_clear(hist, tot, lane)
        (dh, rh, th), (dl, rl, tl) = _find2(tot, rh, rl, th, lane)
        ph = lax.bitwise_or(lax.shift_left(ph, 8), dh)
        plo = lax.bitwise_or(lax.shift_left(plo, 8), dl)

        maxoff = jnp.max(off_v)
        overflow = maxoff > jnp.int32(CAP)

        def _full_levels(args):
            ph, plo, rh, rl, th, tl = args
            for shift in (8, 0):
                @plsc.parallel_loop(0, CHUNKS, step=1, unroll=UN)
                def _(c, shift=shift, ph=ph, plo=plo):
                    sl = pl.ds(c * L, L)
                    u = lax.bitcast_convert_type(buf[sl], jnp.int32)
                    pref = _srl(u, shift + 8)
                    mh = pref == ph
                    ml = pref == plo
                    d = lax.bitwise_and(_srl(u, shift), jnp.int32(0xFF))
                    rep = lax.bitwise_and(c + _srl(c, 3), R - 1) * HIST_WORDS
                    val = (jnp.where(mh, jnp.int32(1), jnp.int32(0))
                           + jnp.where(ml, jnp.int32(65536), jnp.int32(0)))
                    plsc.addupdate_scatter(
                        hist, [rep + d * BSTR + lane], val,
                        mask=jnp.logical_or(mh, ml))
                _scan_and_clear(hist, tot, lane)
                (dh, rh, th), (dl, rl, tl) = _find2(tot, rh, rl, th, lane)
                ph = lax.bitwise_or(lax.shift_left(ph, 8), dh)
                plo = lax.bitwise_or(lax.shift_left(plo, 8), dl)
            return ph, plo

        def _cand_levels(args):
            ph, plo, rh, rl, th, tl = args
            for shift in (8, 0):
                def cbody(s, _, shift=shift, ph=ph, plo=plo):
                    u = plsc.load_gather(cand, [lane * CSTR + s])
                    valid = off_v > s
                    pref = _srl(u, shift + 8)
                    mh = jnp.logical_and(pref == ph, valid)
                    ml = jnp.logical_and(pref == plo, valid)
                    d = lax.bitwise_and(_srl(u, shift), jnp.int32(0xFF))
                    val = (jnp.where(mh, jnp.int32(1), jnp.int32(0))
                           + jnp.where(ml, jnp.int32(65536), jnp.int32(0)))
                    plsc.addupdate_scatter(
                        hist, [d * BSTR + lane], val,
                        mask=jnp.logical_or(mh, ml))
                    return 0
                lax.fori_loop(0, maxoff, cbody, 0)
                _scan_and_clear_r1(hist, tot, lane)
                (dh, rh, th), (dl, rl, tl) = _find2(tot, rh, rl, th, lane)
                ph = lax.bitwise_or(lax.shift_left(ph, 8), dh)
                plo = lax.bitwise_or(lax.shift_left(plo, 8), dl)
            return ph, plo

        ph, plo = lax.cond(overflow, _full_levels, _cand_levels,
                           (ph, plo, rh, rl, th, tl))

        s_hi = lax.bitwise_xor(ph, _topbit())
        s_lo = lax.bitwise_xor(plo, _topbit())

        @plsc.parallel_loop(0, CHUNKS, step=1, unroll=UN)
        def _(c):
            sl = pl.ds(c * L, L)
            u = lax.bitcast_convert_type(buf[sl], jnp.int32)
            s = lax.bitwise_xor(u, _topbit())
            keep = jnp.logical_and(s < s_hi, s > s_lo)
            xv = lax.bitcast_convert_type(_inverse(u), jnp.float32)
            buf[sl] = jnp.where(keep, xv, jnp.float32(0.0))

        pltpu.sync_copy(buf, out_hbm.at[r])
        return 0

    lax.fori_loop(0, ROWS_PER_W, row_body, 0)


def kernel(x, weight):
    return _trunc_kernel(x, weight)


# speedup vs baseline: 1.0223x; 1.0223x over previous
import functools

import jax
import jax.numpy as jnp
from jax import lax
from jax.experimental import pallas as pl
from jax.experimental.pallas import tpu as pltpu
from jax.experimental.pallas import tpu_sc as plsc

B = 128
N = 32768
K = 256

NC = 2
NS = 16
L = 16
NW = NC * NS
ROWS_PER_W = B // NW
CHUNKS = N // L
NB = 256
R = 8
UN = 8
BSTR = L + 1
HIST_WORDS = NB * BSTR
HIST_TOTAL = R * HIST_WORDS
CAP = 1024
CSTR = CAP + 1


def _topbit():
    return jnp.int32(-2**31)


def _monotone(bits):
    m = lax.shift_right_arithmetic(bits, 31)
    flip = lax.bitwise_or(_topbit(), lax.bitwise_and(m, jnp.int32(0x7FFFFFFF)))
    return lax.bitwise_xor(bits, flip)


def _inverse(u):
    m = lax.shift_right_arithmetic(u, 31)
    flip = lax.bitwise_or(
        _topbit(), lax.bitwise_and(lax.bitwise_not(m), jnp.int32(0x7FFFFFFF)))
    return lax.bitwise_xor(u, flip)


def _srl(v, k):
    return lax.shift_right_logical(v, k)


def _scan_and_clear(hist_ref, tot_ref, lane):
    zero = jnp.zeros((L,), jnp.int32)
    @plsc.parallel_loop(0, NB // L, step=1, unroll=2)
    def _(c):
        base = c * L
        acc = jnp.zeros((L,), jnp.int32)
        for rep in range(R):
            for l in range(L):
                idx = rep * HIST_WORDS + (base + lane) * BSTR + l
                acc = acc + plsc.load_gather(hist_ref, [idx])
                plsc.store_scatter(hist_ref, [idx], zero)
        tot_ref[pl.ds(base, L)] = acc


def _scan_and_clear_r1(hist_ref, tot_ref, lane):
    zero = jnp.zeros((L,), jnp.int32)
    @plsc.parallel_loop(0, NB // L, step=1, unroll=2)
    def _(c):
        base = c * L
        acc = jnp.zeros((L,), jnp.int32)
        for l in range(L):
            idx = (base + lane) * BSTR + l
            acc = acc + plsc.load_gather(hist_ref, [idx])
            plsc.store_scatter(hist_ref, [idx], zero)
        tot_ref[pl.ds(base, L)] = acc


def _find2(tot_ref, kr_h, kr_l, m_h, lane):
    def body(c, carry):
        cum_h, cum_l, bsh, rsh, tsh, bsl, rsl, tsl = carry
        tword = tot_ref[pl.ds(c * L, L)]
        t_h = lax.bitwise_and(tword, jnp.int32(0xFFFF))
        t_l = _srl(tword, 16)
        cs_h = cum_h + jnp.cumsum(t_h)
        cs_l = cum_l + jnp.cumsum(t_l)
        a = m_h - cs_h
        hit_h = jnp.logical_and(a < kr_h, a + t_h >= kr_h)
        cv = cs_l - t_l
        hit_l = jnp.logical_and(cv < kr_l, cv + t_l >= kr_l)
        ids = c * L + lane
        zero = jnp.int32(0)
        bsh = bsh + jnp.sum(jnp.where(hit_h, ids + 1, zero))
        rsh = rsh + jnp.sum(jnp.where(hit_h, kr_h - a, zero))
        tsh = tsh + jnp.sum(jnp.where(hit_h, t_h, zero))
        bsl = bsl + jnp.sum(jnp.where(hit_l, ids + 1, zero))
        rsl = rsl + jnp.sum(jnp.where(hit_l, kr_l - cv, zero))
        tsl = tsl + jnp.sum(jnp.where(hit_l, t_l, zero))
        return (cum_h + jnp.sum(t_h), cum_l + jnp.sum(t_l),
                bsh, rsh, tsh, bsl, rsl, tsl)
    z = jnp.int32(0)
    out = lax.fori_loop(0, NB // L, body, (z,) * 8)
    (_, _, bsh, rsh, tsh, bsl, rsl, tsl) = out
    return (bsh - 1, rsh, tsh), (bsl - 1, rsl, tsl)


_mesh = plsc.VectorSubcoreMesh(
    core_axis_name="c", subcore_axis_name="s", num_cores=NC, num_subcores=NS)


@functools.partial(
    pl.kernel,
    out_type=jax.ShapeDtypeStruct((B, N), jnp.float32),
    mesh=_mesh,
    compiler_params=pltpu.CompilerParams(needs_layout_passes=False),
    scratch_types=[
        pltpu.VMEM((N,), jnp.float32),
        pltpu.VMEM((N,), jnp.float32),
        pltpu.VMEM((HIST_TOTAL,), jnp.int32),
        pltpu.VMEM((NB,), jnp.int32),
        pltpu.VMEM((L * CSTR,), jnp.int32),
        pltpu.SemaphoreType.DMA,
        pltpu.SemaphoreType.DMA,
    ],
)
def _trunc_kernel(x_hbm, w_hbm, out_hbm, buf, w_ref, hist, tot, cand,
                  sem_a, sem_b):
    wid = lax.axis_index("s") * NC + lax.axis_index("c")
    lane = lax.iota(jnp.int32, L)
    kk = jnp.int32(K)
    p1val = jnp.full((L,), 65537, jnp.int32)

    pltpu.sync_copy(w_hbm, w_ref)

    zero16 = jnp.zeros((L,), jnp.int32)
    @plsc.parallel_loop(0, HIST_TOTAL // L, step=1, unroll=8)
    def _(c):
        hist[pl.ds(c * L, L)] = zero16

    def row_body(i, _):
        r = wid * ROWS_PER_W + i
        half = N // 2
        in0 = pltpu.async_copy(
            x_hbm.at[r].at[pl.ds(0, half)], buf.at[pl.ds(0, half)], sem_a)
        in1 = pltpu.async_copy(
            x_hbm.at[r].at[pl.ds(half, half)], buf.at[pl.ds(half, half)],
            sem_b)

        def p1_range(lo, hi):
            @plsc.parallel_loop(lo, hi, step=1, unroll=UN)
            def _(c):
                sl = pl.ds(c * L, L)
                xv = buf[sl] * w_ref[sl]
                u = _monotone(lax.bitcast_convert_type(xv, jnp.int32))
                buf[sl] = lax.bitcast_convert_type(u, jnp.float32)
                d = _srl(u, 24)
                rep = lax.bitwise_and(c + _srl(c, 3), R - 1) * HIST_WORDS
                plsc.addupdate_scatter(hist, [rep + d * BSTR + lane], p1val)
        in0.wait()
        p1_range(0, CHUNKS // 2)
        in1.wait()
        p1_range(CHUNKS // 2, CHUNKS)

        _scan_and_clear(hist, tot, lane)
        (ph, rh, th), (plo, rl, tl) = _find2(tot, kk, kk, jnp.int32(N), lane)

        @plsc.parallel_loop(0, CHUNKS, step=1, unroll=UN,
                            carry=jnp.zeros((L,), jnp.int32))
        def off_v(c, off, ph=ph, plo=plo):
            sl = pl.ds(c * L, L)
            u = lax.bitcast_convert_type(buf[sl], jnp.int32)
            pref = _srl(u, 24)
            mh = pref == ph
            ml = pref == plo
            m = jnp.logical_or(mh, ml)
            d = lax.bitwise_and(_srl(u, 16), jnp.int32(0xFF))
            rep = lax.bitwise_and(c + _srl(c, 3), R - 1) * HIST_WORDS
            val = (jnp.where(mh, jnp.int32(1), jnp.int32(0))
                   + jnp.where(ml, jnp.int32(65536), jnp.int32(0)))
            plsc.addupdate_scatter(
                hist, [rep + d * BSTR + lane], val, mask=m)
            plsc.store_scatter(
                cand, [lane * CSTR + off], u,
                mask=jnp.logical_and(m, off < jnp.int32(CAP)))
            return off + jnp.where(m, jnp.int32(1), jnp.int32(0))

        _scan_and_clear(hist, tot, lane)
        (dh, rh, th), (dl, rl, tl) = _find2(tot, rh, rl, th, lane)
        ph = lax.bitwise_or(lax.shift_left(ph, 8), dh)
        plo = lax.bitwise_or(lax.shift_left(plo, 8), dl)

        maxoff = jnp.max(off_v)
        overflow = maxoff > jnp.int32(CAP)

        def _full_levels(args):
            ph, plo, rh, rl, th, tl = args
            for shift in (8, 0):
                @plsc.parallel_loop(0, CHUNKS, step=1, unroll=UN)
                def _(c, shift=shift, ph=ph, plo=plo):
                    sl = pl.ds(c * L, L)
                    u = lax.bitcast_convert_type(buf[sl], jnp.int32)
                    pref = _srl(u, shift + 8)
                    mh = pref == ph
                    ml = pref == plo
                    d = lax.bitwise_and(_srl(u, shift), jnp.int32(0xFF))
                    rep = lax.bitwise_and(c + _srl(c, 3), R - 1) * HIST_WORDS
                    val = (jnp.where(mh, jnp.int32(1), jnp.int32(0))
                           + jnp.where(ml, jnp.int32(65536), jnp.int32(0)))
                    plsc.addupdate_scatter(
                        hist, [rep + d * BSTR + lane], val,
                        mask=jnp.logical_or(mh, ml))
                _scan_and_clear(hist, tot, lane)
                (dh, rh, th), (dl, rl, tl) = _find2(tot, rh, rl, th, lane)
                ph = lax.bitwise_or(lax.shift_left(ph, 8), dh)
                plo = lax.bitwise_or(lax.shift_left(plo, 8), dl)
            return ph, plo

        def _cand_levels(args):
            ph, plo, rh, rl, th, tl = args
            for shift in (8, 0):
                def cbody(s, _, shift=shift, ph=ph, plo=plo):
                    u = plsc.load_gather(cand, [lane * CSTR + s])
                    valid = off_v > s
                    pref = _srl(u, shift + 8)
                    mh = jnp.logical_and(pref == ph, valid)
                    ml = jnp.logical_and(pref == plo, valid)
                    d = lax.bitwise_and(_srl(u, shift), jnp.int32(0xFF))
                    val = (jnp.where(mh, jnp.int32(1), jnp.int32(0))
                           + jnp.where(ml, jnp.int32(65536), jnp.int32(0)))
                    plsc.addupdate_scatter(
                        hist, [d * BSTR + lane], val,
                        mask=jnp.logical_or(mh, ml))
                    return 0
                lax.fori_loop(0, maxoff, cbody, 0)
                _scan_and_clear_r1(hist, tot, lane)
                (dh, rh, th), (dl, rl, tl) = _find2(tot, rh, rl, th, lane)
                ph = lax.bitwise_or(lax.shift_left(ph, 8), dh)
                plo = lax.bitwise_or(lax.shift_left(plo, 8), dl)
            return ph, plo

        ph, plo = lax.cond(overflow, _full_levels, _cand_levels,
                           (ph, plo, rh, rl, th, tl))

        s_hi = lax.bitwise_xor(ph, _topbit())
        s_lo = lax.bitwise_xor(plo, _topbit())

        def p5_range(lo, hi):
            @plsc.parallel_loop(lo, hi, step=1, unroll=UN)
            def _(c):
                sl = pl.ds(c * L, L)
                u = lax.bitcast_convert_type(buf[sl], jnp.int32)
                s = lax.bitwise_xor(u, _topbit())
                keep = jnp.logical_and(s < s_hi, s > s_lo)
                xv = lax.bitcast_convert_type(_inverse(u), jnp.float32)
                buf[sl] = jnp.where(keep, xv, jnp.float32(0.0))
        p5_range(0, CHUNKS // 2)
        out0 = pltpu.async_copy(
            buf.at[pl.ds(0, half)], out_hbm.at[r].at[pl.ds(0, half)], sem_a)
        p5_range(CHUNKS // 2, CHUNKS)
        out1 = pltpu.async_copy(
            buf.at[pl.ds(half, half)], out_hbm.at[r].at[pl.ds(half, half)],
            sem_b)
        out0.wait()
        out1.wait()
        return 0

    lax.fori_loop(0, ROWS_PER_W, row_body, 0)


def kernel(x, weight):
    return _trunc_kernel(x, weight)


# speedup vs baseline: 1.0431x; 1.0204x over previous
import functools

import jax
import jax.numpy as jnp
from jax import lax
from jax.experimental import pallas as pl
from jax.experimental.pallas import tpu as pltpu
from jax.experimental.pallas import tpu_sc as plsc

B = 128
N = 32768
K = 256

NC = 2
NS = 16
L = 16
NW = NC * NS
ROWS_PER_W = B // NW
CHUNKS = N // L
NB = 256
R = 8
UN = 8
BSTR = L + 1
HIST_WORDS = NB * BSTR
HIST_TOTAL = R * HIST_WORDS
CAP = 1024
CSTR = CAP + 1


def _topbit():
    return jnp.int32(-2**31)


def _monotone(bits):
    m = lax.shift_right_arithmetic(bits, 31)
    flip = lax.bitwise_or(_topbit(), lax.bitwise_and(m, jnp.int32(0x7FFFFFFF)))
    return lax.bitwise_xor(bits, flip)


def _inverse(u):
    m = lax.shift_right_arithmetic(u, 31)
    flip = lax.bitwise_or(
        _topbit(), lax.bitwise_and(lax.bitwise_not(m), jnp.int32(0x7FFFFFFF)))
    return lax.bitwise_xor(u, flip)


def _srl(v, k):
    return lax.shift_right_logical(v, k)


def _scan_and_clear(hist_ref, tot_ref, lane):
    zero = jnp.zeros((L,), jnp.int32)
    @plsc.parallel_loop(0, NB // L, step=1, unroll=2)
    def _(c):
        base = c * L
        acc = jnp.zeros((L,), jnp.int32)
        for rep in range(R):
            for l in range(L):
                idx = rep * HIST_WORDS + (base + lane) * BSTR + l
                acc = acc + plsc.load_gather(hist_ref, [idx])
                plsc.store_scatter(hist_ref, [idx], zero)
        tot_ref[pl.ds(base, L)] = acc


def _scan_and_clear_r1(hist_ref, tot_ref, lane):
    zero = jnp.zeros((L,), jnp.int32)
    @plsc.parallel_loop(0, NB // L, step=1, unroll=2)
    def _(c):
        base = c * L
        acc = jnp.zeros((L,), jnp.int32)
        for l in range(L):
            idx = (base + lane) * BSTR + l
            acc = acc + plsc.load_gather(hist_ref, [idx])
            plsc.store_scatter(hist_ref, [idx], zero)
        tot_ref[pl.ds(base, L)] = acc


def _find2(tot_ref, kr_h, kr_l, m_h, lane):
    def body(c, carry):
        cum_h, cum_l, bsh, rsh, tsh, bsl, rsl, tsl = carry
        tword = tot_ref[pl.ds(c * L, L)]
        t_h = lax.bitwise_and(tword, jnp.int32(0xFFFF))
        t_l = _srl(tword, 16)
        cs_h = cum_h + jnp.cumsum(t_h)
        cs_l = cum_l + jnp.cumsum(t_l)
        a = m_h - cs_h
        hit_h = jnp.logical_and(a < kr_h, a + t_h >= kr_h)
        cv = cs_l - t_l
        hit_l = jnp.logical_and(cv < kr_l, cv + t_l >= kr_l)
        ids = c * L + lane
        zero = jnp.int32(0)
        bsh = bsh + jnp.sum(jnp.where(hit_h, ids + 1, zero))
        rsh = rsh + jnp.sum(jnp.where(hit_h, kr_h - a, zero))
        tsh = tsh + jnp.sum(jnp.where(hit_h, t_h, zero))
        bsl = bsl + jnp.sum(jnp.where(hit_l, ids + 1, zero))
        rsl = rsl + jnp.sum(jnp.where(hit_l, kr_l - cv, zero))
        tsl = tsl + jnp.sum(jnp.where(hit_l, t_l, zero))
        return (cum_h + jnp.sum(t_h), cum_l + jnp.sum(t_l),
                bsh, rsh, tsh, bsl, rsl, tsl)
    z = jnp.int32(0)
    out = lax.fori_loop(0, NB // L, body, (z,) * 8)
    (_, _, bsh, rsh, tsh, bsl, rsl, tsl) = out
    return (bsh - 1, rsh, tsh), (bsl - 1, rsl, tsl)


_mesh = plsc.VectorSubcoreMesh(
    core_axis_name="c", subcore_axis_name="s", num_cores=NC, num_subcores=NS)


@functools.partial(
    pl.kernel,
    out_type=jax.ShapeDtypeStruct((B, N), jnp.float32),
    mesh=_mesh,
    compiler_params=pltpu.CompilerParams(needs_layout_passes=False),
    scratch_types=[
        pltpu.VMEM((N,), jnp.float32),
        pltpu.VMEM((N,), jnp.float32),
        pltpu.VMEM((HIST_TOTAL,), jnp.int32),
        pltpu.VMEM((NB,), jnp.int32),
        pltpu.VMEM((L * CSTR,), jnp.int32),
    ],
)
def _trunc_kernel(x_hbm, w_hbm, out_hbm, buf, w_ref, hist, tot, cand):
    wid = lax.axis_index("s") * NC + lax.axis_index("c")
    lane = lax.iota(jnp.int32, L)
    kk = jnp.int32(K)
    p1val = jnp.full((L,), 65537, jnp.int32)

    pltpu.sync_copy(w_hbm, w_ref)

    zero16 = jnp.zeros((L,), jnp.int32)
    @plsc.parallel_loop(0, HIST_TOTAL // L, step=1, unroll=8)
    def _(c):
        hist[pl.ds(c * L, L)] = zero16

    def row_body(i, _):
        r = wid * ROWS_PER_W + i
        pltpu.sync_copy(x_hbm.at[r], buf)

        @plsc.parallel_loop(0, CHUNKS, step=1, unroll=UN)
        def _(c):
            sl = pl.ds(c * L, L)
            xv = buf[sl] * w_ref[sl]
            u = _monotone(lax.bitcast_convert_type(xv, jnp.int32))
            buf[sl] = lax.bitcast_convert_type(u, jnp.float32)
            d = _srl(u, 24)
            rep = lax.bitwise_and(c + _srl(c, 3), R - 1) * HIST_WORDS
            plsc.addupdate_scatter(hist, [rep + d * BSTR + lane], p1val)

        _scan_and_clear(hist, tot, lane)
        (ph, rh, th), (plo, rl, tl) = _find2(tot, kk, kk, jnp.int32(N), lane)

        @plsc.parallel_loop(0, CHUNKS, step=1, unroll=UN,
                            carry=jnp.zeros((L,), jnp.int32))
        def off_v(c, off, ph=ph, plo=plo):
            sl = pl.ds(c * L, L)
            u = lax.bitcast_convert_type(buf[sl], jnp.int32)
            pref = _srl(u, 24)
            mh = pref == ph
            ml = pref == plo
            m = jnp.logical_or(mh, ml)
            d = lax.bitwise_and(_srl(u, 16), jnp.int32(0xFF))
            rep = lax.bitwise_and(c + _srl(c, 3), R - 1) * HIST_WORDS
            val = (jnp.where(mh, jnp.int32(1), jnp.int32(0))
                   + jnp.where(ml, jnp.int32(65536), jnp.int32(0)))
            plsc.addupdate_scatter(
                hist, [rep + d * BSTR + lane], val, mask=m)
            plsc.store_scatter(
                cand, [lane * CSTR + off], u,
                mask=jnp.logical_and(m, off < jnp.int32(CAP)))
            return off + jnp.where(m, jnp.int32(1), jnp.int32(0))

        _scan_and_clear(hist, tot, lane)
        (dh, rh, th), (dl, rl, tl) = _find2(tot, rh, rl, th, lane)
        ph = lax.bitwise_or(lax.shift_left(ph, 8), dh)
        plo = lax.bitwise_or(lax.shift_left(plo, 8), dl)

        maxoff = jnp.max(off_v)
        overflow = maxoff > jnp.int32(CAP)

        def _full_levels(args):
            ph, plo, rh, rl, th, tl = args
            for shift in (8, 0):
                @plsc.parallel_loop(0, CHUNKS, step=1, unroll=UN)
                def _(c, shift=shift, ph=ph, plo=plo):
                    sl = pl.ds(c * L, L)
                    u = lax.bitcast_convert_type(buf[sl], jnp.int32)
                    pref = _srl(u, shift + 8)
                    mh = pref == ph
                    ml = pref == plo
                    d = lax.bitwise_and(_srl(u, shift), jnp.int32(0xFF))
                    rep = lax.bitwise_and(c + _srl(c, 3), R - 1) * HIST_WORDS
                    val = (jnp.where(mh, jnp.int32(1), jnp.int32(0))
                           + jnp.where(ml, jnp.int32(65536), jnp.int32(0)))
                    plsc.addupdate_scatter(
                        hist, [rep + d * BSTR + lane], val,
                        mask=jnp.logical_or(mh, ml))
                _scan_and_clear(hist, tot, lane)
                (dh, rh, th), (dl, rl, tl) = _find2(tot, rh, rl, th, lane)
                ph = lax.bitwise_or(lax.shift_left(ph, 8), dh)
                plo = lax.bitwise_or(lax.shift_left(plo, 8), dl)
            return ph, plo

        def _cand_levels(args):
            ph, plo, rh, rl, th, tl = args
            for shift in (8, 0):
                def cbody(s, _, shift=shift, ph=ph, plo=plo):
                    u = plsc.load_gather(cand, [lane * CSTR + s])
                    valid = off_v > s
                    pref = _srl(u, shift + 8)
                    mh = jnp.logical_and(pref == ph, valid)
                    ml = jnp.logical_and(pref == plo, valid)
                    d = lax.bitwise_and(_srl(u, shift), jnp.int32(0xFF))
                    val = (jnp.where(mh, jnp.int32(1), jnp.int32(0))
                           + jnp.where(ml, jnp.int32(65536), jnp.int32(0)))
                    plsc.addupdate_scatter(
                        hist, [d * BSTR + lane], val,
                        mask=jnp.logical_or(mh, ml))
                    return 0
                lax.fori_loop(0, maxoff, cbody, 0)
                _scan_and_clear_r1(hist, tot, lane)
                (dh, rh, th), (dl, rl, tl) = _find2(tot, rh, rl, th, lane)
                ph = lax.bitwise_or(lax.shift_left(ph, 8), dh)
                plo = lax.bitwise_or(lax.shift_left(plo, 8), dl)
            return ph, plo

        ph, plo = lax.cond(overflow, _full_levels, _cand_levels,
                           (ph, plo, rh, rl, th, tl))

        s_hi = lax.bitwise_xor(ph, _topbit())
        s_lo = lax.bitwise_xor(plo, _topbit())

        @plsc.parallel_loop(0, CHUNKS, step=1, unroll=UN)
        def _(c):
            sl = pl.ds(c * L, L)
            u = lax.bitcast_convert_type(buf[sl], jnp.int32)
            s = lax.bitwise_xor(u, _topbit())
            keep = jnp.logical_and(s < s_hi, s > s_lo)
            xv = lax.bitcast_convert_type(_inverse(u), jnp.float32)
            buf[sl] = jnp.where(keep, xv, jnp.float32(0.0))

        pltpu.sync_copy(buf, out_hbm.at[r])
        return 0

    lax.fori_loop(0, ROWS_PER_W, row_body, 0)


def kernel(x, weight):
    return _trunc_kernel(x, weight)
